# baseline (speedup 1.0000x reference)
import functools

import jax
import jax.numpy as jnp
from jax import lax
from jax.experimental import pallas as pl
from jax.experimental.pallas import tpu as pltpu
from jax.experimental.pallas import tpu_sc as plsc

_NC = 2
_NS = 16
_NW = _NC * _NS
_CHUNK = 128


def _make_sc_refine(B):
  per_w = B // _NW
  n_chunks = per_w // _CHUNK
  n_rows = 6 * n_chunks
  n_groups = per_w // 16

  mesh = plsc.VectorSubcoreMesh(core_axis_name="c", subcore_axis_name="s")

  @functools.partial(
      pl.kernel,
      mesh=mesh,
      compiler_params=pltpu.CompilerParams(needs_layout_passes=False),
      out_type=jax.ShapeDtypeStruct((B * 16,), jnp.float32),
      scratch_types=[
          pltpu.VMEM((per_w,), jnp.int32),
          pltpu.VMEM((n_rows, _CHUNK), jnp.int32),
          pltpu.VMEM((6, per_w), jnp.float32),
          pltpu.VMEM((per_w * 16,), jnp.float32),
          pltpu.VMEM((per_w * 16,), jnp.float32),
          pltpu.SemaphoreType.DMA,
      ],
  )
  def body(poses_hbm, idx_hbm, table_hbm, out_hbm,
           raw_v, idx_v, soa_v, pin_v, pout_v, sem):
    wid = lax.axis_index("s") * _NC + lax.axis_index("c")
    base = wid * per_w

    pltpu.sync_copy(idx_hbm.at[pl.ds(base, per_w)], raw_v)

    def expand(v, carry):
      src = raw_v[pl.ds(v * 16, 16)] * 6
      j = v // 8
      l = (v % 8) * 16
      for c in range(6):
        idx_v[c * n_chunks + j, pl.ds(l, 16)] = src + c
      return carry

    lax.fori_loop(0, per_w // 16, expand, 0, unroll=8)

    gathers = [
        pltpu.async_copy(table_hbm.at[idx_v.at[c * n_chunks + j]],
                         soa_v.at[c, pl.ds(j * _CHUNK, _CHUNK)], sem)
        for c in range(6)
        for j in range(n_chunks)
    ]
    pltpu.sync_copy(poses_hbm.at[pl.ds(base * 16, per_w * 16)], pin_v)
    for g in gathers:
      g.wait()

    lanes = lax.iota(jnp.int32, 16)

    def compute(g, carry):
      rbase = (g * 16 + lanes) * 16

      w0 = soa_v[0, pl.ds(g * 16, 16)]
      w1 = soa_v[1, pl.ds(g * 16, 16)]
      w2 = soa_v[2, pl.ds(g * 16, 16)]
      t = [soa_v[3 + i, pl.ds(g * 16, 16)] for i in range(3)]
      p = [plsc.load_gather(pin_v, [rbase + e]) for e in range(16)]

      th2 = w0 * w0 + w1 * w1 + w2 * w2
      seed = jnp.int32(0x5F3759DF) - (plsc.bitcast(th2, jnp.int32) >> 1)
      y = plsc.bitcast(seed, jnp.float32)
      for _ in range(3):
        y = y * (1.5 - 0.5 * th2 * y * y)
      theta = th2 * y
      safe = jnp.maximum(theta, 1e-8)
      inv = 1.0 / safe
      k0 = w0 * inv
      k1 = w1 * inv
      k2 = w2 * inv
      s = theta * (1.0 - th2 * (1.0 / 6.0) *
                   (1.0 - th2 * (1.0 / 20.0) *
                    (1.0 - th2 * (1.0 / 42.0) *
                     (1.0 - th2 * (1.0 / 72.0)))))
      oc = th2 * 0.5 * (1.0 - th2 * (1.0 / 12.0) *
                        (1.0 - th2 * (1.0 / 30.0) *
                         (1.0 - th2 * (1.0 / 56.0))))

      kk01 = k0 * k1
      kk02 = k0 * k2
      kk12 = k1 * k2
      sq0 = k0 * k0
      sq1 = k1 * k1
      sq2 = k2 * k2
      d = [
          [1.0 - oc * (sq1 + sq2), oc * kk01 - s * k2, oc * kk02 + s * k1],
          [oc * kk01 + s * k2, 1.0 - oc * (sq0 + sq2), oc * kk12 - s * k0],
          [oc * kk02 - s * k1, oc * kk12 + s * k0, 1.0 - oc * (sq0 + sq1)],
      ]

      out_e = [None] * 16
      for i in range(3):
        for j in range(3):
          out_e[4 * i + j] = (d[i][0] * p[j] + d[i][1] * p[4 + j] +
                              d[i][2] * p[8 + j])
        out_e[4 * i + 3] = p[4 * i + 3] + t[i]
      for e in range(12, 16):
        out_e[e] = p[e]
      for e in range(16):
        plsc.store_scatter(pout_v, [rbase + e], out_e[e])
      return carry

    lax.fori_loop(0, n_groups, compute, 0)
    pltpu.sync_copy(pout_v, out_hbm.at[pl.ds(base * 16, per_w * 16)])

  return body


def kernel(orig_poses, idx, pose_refinements):
  B = orig_poses.shape[0]
  poses_flat = orig_poses.reshape(B * 16)
  table_flat = pose_refinements.reshape(-1)
  out_flat = _make_sc_refine(B)(poses_flat, idx.astype(jnp.int32), table_flat)
  return out_flat.reshape(B, 4, 4)


# speedup vs baseline: 1.1425x; 1.1425x over previous
import functools

import jax
import jax.numpy as jnp
from jax import lax
from jax.experimental import pallas as pl
from jax.experimental.pallas import tpu as pltpu
from jax.experimental.pallas import tpu_sc as plsc

_NC = 2
_NS = 16
_NW = _NC * _NS
_CHUNK = 128


def _make_sc_refine(B):
  per_w = B // _NW
  n_chunks = per_w // _CHUNK
  n_groups = per_w // 16

  mesh = plsc.VectorSubcoreMesh(core_axis_name="c", subcore_axis_name="s")

  @functools.partial(
      pl.kernel,
      mesh=mesh,
      compiler_params=pltpu.CompilerParams(needs_layout_passes=False),
      out_type=jax.ShapeDtypeStruct((16, B), jnp.float32),
      scratch_types=[
          pltpu.VMEM((n_chunks, _CHUNK), jnp.int32),
          pltpu.VMEM((6 * n_chunks, _CHUNK), jnp.int32),
          pltpu.VMEM((6, per_w), jnp.float32),
          pltpu.VMEM((16, per_w), jnp.float32),
          pltpu.VMEM((16, per_w), jnp.float32),
          pltpu.SemaphoreType.DMA,
      ],
  )
  def body(poses_hbm, idx_hbm, table_hbm, out_hbm,
           idx_v, idx6_v, soa_v, pin_v, pout_v, sem):
    wid = lax.axis_index("s") * _NC + lax.axis_index("c")
    base = wid * per_w

    for j in range(n_chunks):
      pltpu.sync_copy(idx_hbm.at[pl.ds(base + j * _CHUNK, _CHUNK)],
                      idx_v.at[j])

    def expand(v, carry):
      j = v // 8
      l = (v % 8) * 16
      src = idx_v[j, pl.ds(l, 16)] * 6
      for c in range(6):
        idx6_v[c * n_chunks + j, pl.ds(l, 16)] = src + c
      return carry

    lax.fori_loop(0, per_w // 16, expand, 0, unroll=8)

    gathers = [
        pltpu.async_copy(table_hbm.at[idx6_v.at[c * n_chunks + j]],
                         soa_v.at[c, pl.ds(j * _CHUNK, _CHUNK)], sem)
        for c in range(6)
        for j in range(n_chunks)
    ]
    pltpu.sync_copy(poses_hbm.at[:, pl.ds(base, per_w)], pin_v)
    for g in gathers:
      g.wait()

    def compute(g, carry):
      sl = pl.ds(g * 16, 16)

      w0 = soa_v[0, sl]
      w1 = soa_v[1, sl]
      w2 = soa_v[2, sl]
      t = [soa_v[3 + i, sl] for i in range(3)]
      p = [pin_v[e, sl] for e in range(16)]

      th2 = w0 * w0 + w1 * w1 + w2 * w2
      seed = jnp.int32(0x5F3759DF) - (plsc.bitcast(th2, jnp.int32) >> 1)
      y = plsc.bitcast(seed, jnp.float32)
      for _ in range(3):
        y = y * (1.5 - 0.5 * th2 * y * y)
      theta = th2 * y
      safe = jnp.maximum(theta, 1e-8)
      inv = 1.0 / safe
      k0 = w0 * inv
      k1 = w1 * inv
      k2 = w2 * inv
      s = theta * (1.0 - th2 * (1.0 / 6.0) *
                   (1.0 - th2 * (1.0 / 20.0) *
                    (1.0 - th2 * (1.0 / 42.0) *
                     (1.0 - th2 * (1.0 / 72.0)))))
      oc = th2 * 0.5 * (1.0 - th2 * (1.0 / 12.0) *
                        (1.0 - th2 * (1.0 / 30.0) *
                         (1.0 - th2 * (1.0 / 56.0))))

      kk01 = k0 * k1
      kk02 = k0 * k2
      kk12 = k1 * k2
      sq0 = k0 * k0
      sq1 = k1 * k1
      sq2 = k2 * k2
      d = [
          [1.0 - oc * (sq1 + sq2), oc * kk01 - s * k2, oc * kk02 + s * k1],
          [oc * kk01 + s * k2, 1.0 - oc * (sq0 + sq2), oc * kk12 - s * k0],
          [oc * kk02 - s * k1, oc * kk12 + s * k0, 1.0 - oc * (sq0 + sq1)],
      ]

      for i in range(3):
        for j in range(3):
          pout_v[4 * i + j, sl] = (d[i][0] * p[j] + d[i][1] * p[4 + j] +
                                   d[i][2] * p[8 + j])
        pout_v[4 * i + 3, sl] = p[4 * i + 3] + t[i]
      for e in range(12, 16):
        pout_v[e, sl] = p[e]
      return carry

    lax.fori_loop(0, n_groups, compute, 0)
    pltpu.sync_copy(pout_v, out_hbm.at[:, pl.ds(base, per_w)])

  return body


def kernel(orig_poses, idx, pose_refinements):
  B = orig_poses.shape[0]
  poses16 = orig_poses.transpose(1, 2, 0).reshape(16, B)
  out16 = _make_sc_refine(B)(poses16, idx.astype(jnp.int32),
                             pose_refinements.reshape(-1))
  return out16.reshape(4, 4, B).transpose(2, 0, 1)
